# baseline (speedup 1.0000x reference)
import functools

import jax
import jax.numpy as jnp
from jax import lax
from jax.experimental import pallas as pl
from jax.experimental.pallas import tpu as pltpu
from jax.experimental.pallas import tpu_sc as plsc

_NC = 2
_NS = 16
_NW = _NC * _NS

_CH = 128
_K = 2
_NBUF = 2
_L = 16


_TW = 1024


def _repack(emb_t):
    Dd, V = emb_t.shape
    G = -(-V // _TW)

    def body(x_ref, o_ref):
        y = x_ref[...].T.reshape(_TW // 4, 4, Dd)
        for m in range(4):
            o_ref[:, m * Dd:(m + 1) * Dd] = y[:, m, :]

    return pl.pallas_call(
        body,
        grid=(G,),
        in_specs=[pl.BlockSpec((Dd, _TW), lambda g: (0, g))],
        out_specs=pl.BlockSpec((_TW // 4, 4 * Dd), lambda g: (g, 0)),
        out_shape=jax.ShapeDtypeStruct((G * _TW // 4, 4 * Dd), jnp.float32),
    )(emb_t)


def kernel(inputs, embedding):
    B, H = inputs.shape
    V, D = embedding.shape
    N = B * H
    assert N % (_NW * _CH * _K) == 0 and D % 8 == 0 and B % _CH == 0
    assert V % 4 == 0
    n_per_w = N // _NW
    n_ch = n_per_w // _CH
    n_grp = n_ch // _K
    nb = B // _CH
    nd = D // 8
    w4 = 4 * D
    assert n_grp % _NBUF == 0

    idx = inputs.T.reshape(_NW, n_ch, _CH)
    mesh = plsc.VectorSubcoreMesh(core_axis_name="c", subcore_axis_name="s")

    @functools.partial(
        pl.kernel,
        out_type=jax.ShapeDtypeStruct((H, nd, nb, 8, _CH), jnp.float32),
        mesh=mesh,
        compiler_params=pltpu.CompilerParams(
            use_tc_tiling_on_sc=True, needs_layout_passes=False
        ),
        scratch_types=[
            pltpu.VMEM((n_ch, _CH), jnp.int32),
            pltpu.VMEM((n_ch, _CH), jnp.int32),
            pltpu.VMEM((_NBUF, _K * _CH, w4), jnp.float32),
            pltpu.VMEM((_NBUF, _K, nd, 8, _CH), jnp.float32),
            pltpu.SemaphoreType.DMA,
            pltpu.SemaphoreType.DMA,
        ],
    )
    def _embed(idx_hbm, tab_hbm, out_hbm, q_v, m_v, rows_v, tiles_v, gsem, ssem):
        wid = lax.axis_index("s") * _NC + lax.axis_index("c")
        ch0 = wid * n_ch
        pltpu.sync_copy(idx_hbm.at[wid], q_v)

        @pl.loop(0, n_ch)
        def _idx_prep(ch):
            for cb in range(_CH // _L):
                raw = q_v[ch, pl.ds(cb * _L, _L)]
                m_v[ch, pl.ds(cb * _L, _L)] = (raw & 3) * D
                q_v[ch, pl.ds(cb * _L, _L)] = lax.shift_right_logical(raw, 2)

        def fire(grp, buf):
            for t in range(_K):
                ch = grp * _K + t
                pltpu.async_copy(
                    tab_hbm.at[q_v.at[ch]],
                    rows_v.at[buf, pl.ds(t * _CH, _CH)],
                    gsem,
                )

        def drain(grp, buf):
            for t in range(_K):
                ch = grp * _K + t
                pltpu.make_async_copy(
                    tab_hbm.at[q_v.at[ch]],
                    rows_v.at[buf, pl.ds(t * _CH, _CH)],
                    gsem,
                ).wait()

        def out_dma(grp, buf, t):
            gch = ch0 + grp * _K + t
            h = gch // nb
            cb = gch % nb
            return pltpu.make_async_copy(
                tiles_v.at[buf, t],
                out_hbm.at[h, pl.ds(0, nd), cb],
                ssem,
            )

        def assemble_and_store(grp, buf):
            @pl.loop(0, _K)
            def _t_loop(t):
                ch = grp * _K + t
                for cb in range(_CH // _L):
                    c_vec = lax.iota(jnp.int32, _L) + cb * _L + t * _CH
                    m_vec = m_v[ch, pl.ds(cb * _L, _L)]
                    for db in range(nd):
                        for r in range(8):
                            vals = plsc.load_gather(
                                rows_v.at[buf], [c_vec, m_vec + (db * 8 + r)]
                            )
                            tiles_v[buf, t, db, r, pl.ds(cb * _L, _L)] = vals
                out_dma(grp, buf, t).start()

        def drain_store(grp, buf):
            @pl.loop(0, _K)
            def _t_loop(t):
                out_dma(grp, buf, t).wait()

        fire(0, 0)

        @pl.loop(0, n_grp, step=_NBUF)
        def _grp_loop(g0):
            for b in range(_NBUF):
                g = g0 + b

                @pl.when(g + 1 < n_grp)
                def _():
                    fire(g + 1, (b + 1) % _NBUF)

                drain(g, b)

                @pl.when(g >= _NBUF)
                def _():
                    drain_store(g - _NBUF, b)

                assemble_and_store(g, b)

        for b in range(_NBUF):
            drain_store(n_grp - _NBUF + b, b)

    out5 = _embed(idx, _repack(embedding.T))
    return (
        out5.transpose(0, 1, 3, 2, 4)
        .reshape(H, D, B)
        .transpose(2, 0, 1)
    )


# speedup vs baseline: 1.2757x; 1.2757x over previous
import functools

import jax
import jax.numpy as jnp
from jax import lax
from jax.experimental import pallas as pl
from jax.experimental.pallas import tpu as pltpu
from jax.experimental.pallas import tpu_sc as plsc

_NC = 2
_NS = 16
_NW = _NC * _NS

_CH = 128
_K = 2
_NBUF = 2
_L = 16


_TW = 2048


def _repack(emb_t):
    Dd, V = emb_t.shape
    G = -(-V // _TW)

    def body(x_ref, o_ref):
        eye = jnp.eye(Dd, dtype=jnp.float32)
        yt = jax.lax.dot_general(
            x_ref[...], eye, (((0,), (0,)), ((), ())),
            preferred_element_type=jnp.float32,
        )
        y = yt.reshape(_TW // 4, 4, Dd)
        for m in range(4):
            o_ref[:, m * Dd:(m + 1) * Dd] = y[:, m, :]

    return pl.pallas_call(
        body,
        grid=(G,),
        in_specs=[pl.BlockSpec((Dd, _TW), lambda g: (0, g))],
        out_specs=pl.BlockSpec((_TW // 4, 4 * Dd), lambda g: (g, 0)),
        out_shape=jax.ShapeDtypeStruct((G * _TW // 4, 4 * Dd), jnp.float32),
    )(emb_t)


def kernel(inputs, embedding):
    B, H = inputs.shape
    V, D = embedding.shape
    N = B * H
    assert N % (_NW * _CH * _K) == 0 and D % 8 == 0 and B % _CH == 0
    assert V % 4 == 0
    n_per_w = N // _NW
    n_ch = n_per_w // _CH
    n_grp = n_ch // _K
    nb = B // _CH
    nd = D // 8
    w4 = 4 * D
    assert n_grp % _NBUF == 0

    idx = inputs.T.reshape(_NW, n_ch, _CH)
    mesh = plsc.VectorSubcoreMesh(core_axis_name="c", subcore_axis_name="s")

    @functools.partial(
        pl.kernel,
        out_type=jax.ShapeDtypeStruct((H, nd, nb, 8, _CH), jnp.float32),
        mesh=mesh,
        compiler_params=pltpu.CompilerParams(
            use_tc_tiling_on_sc=True, needs_layout_passes=False
        ),
        scratch_types=[
            pltpu.VMEM((n_ch, _CH), jnp.int32),
            pltpu.VMEM((n_ch, _CH), jnp.int32),
            pltpu.VMEM((_NBUF, _K * _CH, w4), jnp.float32),
            pltpu.VMEM((_NBUF, _K, nd, 8, _CH), jnp.float32),
            pltpu.SemaphoreType.DMA,
            pltpu.SemaphoreType.DMA,
        ],
    )
    def _embed(idx_hbm, tab_hbm, out_hbm, q_v, m_v, rows_v, tiles_v, gsem, ssem):
        wid = lax.axis_index("s") * _NC + lax.axis_index("c")
        ch0 = wid * n_ch
        pltpu.sync_copy(idx_hbm.at[wid], q_v)

        @pl.loop(0, n_ch)
        def _idx_prep(ch):
            for cb in range(_CH // _L):
                raw = q_v[ch, pl.ds(cb * _L, _L)]
                m_v[ch, pl.ds(cb * _L, _L)] = (raw & 3) * D
                q_v[ch, pl.ds(cb * _L, _L)] = lax.shift_right_logical(raw, 2)

        def fire(grp, buf):
            for t in range(_K):
                ch = grp * _K + t
                pltpu.async_copy(
                    tab_hbm.at[q_v.at[ch]],
                    rows_v.at[buf, pl.ds(t * _CH, _CH)],
                    gsem,
                )

        def drain(grp, buf):
            for t in range(_K):
                ch = grp * _K + t
                pltpu.make_async_copy(
                    tab_hbm.at[q_v.at[ch]],
                    rows_v.at[buf, pl.ds(t * _CH, _CH)],
                    gsem,
                ).wait()

        def out_dma(grp, buf, t):
            gch = ch0 + grp * _K + t
            h = gch // nb
            cb = gch % nb
            return pltpu.make_async_copy(
                tiles_v.at[buf, t],
                out_hbm.at[h, pl.ds(0, nd), cb],
                ssem,
            )

        def assemble_and_store(grp, buf):
            @pl.loop(0, _K)
            def _t_loop(t):
                ch = grp * _K + t
                for cb in range(_CH // _L):
                    c_vec = lax.iota(jnp.int32, _L) + cb * _L + t * _CH
                    m_vec = m_v[ch, pl.ds(cb * _L, _L)]
                    for db in range(nd):
                        for r in range(8):
                            vals = plsc.load_gather(
                                rows_v.at[buf], [c_vec, m_vec + (db * 8 + r)]
                            )
                            tiles_v[buf, t, db, r, pl.ds(cb * _L, _L)] = vals
                out_dma(grp, buf, t).start()

        def drain_store(grp, buf):
            @pl.loop(0, _K)
            def _t_loop(t):
                out_dma(grp, buf, t).wait()

        fire(0, 0)

        @pl.loop(0, n_grp, step=_NBUF)
        def _grp_loop(g0):
            for b in range(_NBUF):
                g = g0 + b

                @pl.when(g + 1 < n_grp)
                def _():
                    fire(g + 1, (b + 1) % _NBUF)

                drain(g, b)

                @pl.when(g >= _NBUF)
                def _():
                    drain_store(g - _NBUF, b)

                assemble_and_store(g, b)

        for b in range(_NBUF):
            drain_store(n_grp - _NBUF + b, b)

    out5 = _embed(idx, _repack(embedding.T))
    return (
        out5.transpose(0, 1, 3, 2, 4)
        .reshape(H, D, B)
        .transpose(2, 0, 1)
    )


# speedup vs baseline: 1.4420x; 1.1304x over previous
import functools

import jax
import jax.numpy as jnp
from jax import lax
from jax.experimental import pallas as pl
from jax.experimental.pallas import tpu as pltpu
from jax.experimental.pallas import tpu_sc as plsc

_NC = 2
_NS = 16
_NW = _NC * _NS

_CH = 128
_K = 8
_NBUF = 2


def kernel(inputs, embedding):
    B, H = inputs.shape
    V, D = embedding.shape
    N = B * H
    assert N % (_NW * _CH * _K) == 0
    n_per_w = N // _NW
    n_ch = n_per_w // _CH
    n_grp = n_ch // _K
    assert n_grp % _NBUF == 0

    idx = inputs.T.reshape(_NW, n_ch, _CH)
    mesh = plsc.VectorSubcoreMesh(core_axis_name="c", subcore_axis_name="s")

    @functools.partial(
        pl.kernel,
        out_type=jax.ShapeDtypeStruct((N, D), jnp.float32),
        mesh=mesh,
        compiler_params=pltpu.CompilerParams(use_tc_tiling_on_sc=False),
        scratch_types=[
            pltpu.VMEM((n_ch, _CH), jnp.int32),
            pltpu.VMEM((_NBUF, _K * _CH, D), jnp.float32),
            pltpu.SemaphoreType.DMA,
        ],
    )
    def _embed(idx_hbm, tab_hbm, out_hbm, idx_v, rows_v, gsem):
        wid = lax.axis_index("s") * _NC + lax.axis_index("c")
        base = wid * n_per_w
        pltpu.sync_copy(idx_hbm.at[wid], idx_v)

        def fire(grp, buf):
            for t in range(_K):
                ch = grp * _K + t
                pltpu.async_copy(
                    tab_hbm.at[idx_v.at[ch]],
                    rows_v.at[buf, pl.ds(t * _CH, _CH)],
                    gsem,
                )

        def drain(grp, buf):
            for t in range(_K):
                ch = grp * _K + t
                pltpu.make_async_copy(
                    tab_hbm.at[idx_v.at[ch]],
                    rows_v.at[buf, pl.ds(t * _CH, _CH)],
                    gsem,
                ).wait()

        fire(0, 0)

        @pl.loop(0, n_grp, step=_NBUF)
        def _grp_loop(g0):
            for b in range(_NBUF):
                g = g0 + b

                @pl.when(g + 1 < n_grp)
                def _():
                    fire(g + 1, (b + 1) % _NBUF)

                drain(g, b)
                pltpu.sync_copy(
                    rows_v.at[b],
                    out_hbm.at[pl.ds(base + g * _K * _CH, _K * _CH)],
                )

    out = _embed(idx, embedding)
    return out.reshape(H, B, D).transpose(1, 0, 2)
